# Initial kernel scaffold; baseline (speedup 1.0000x reference)
import jax
import jax.numpy as jnp
from jax.experimental import pallas as pl


def kernel(input, edge_index, edge_weight, h_init, W, b):
    raise NotImplementedError("write your pallas kernel here")


# speedup vs baseline: 6.8121x; 6.8121x over previous
import math

import jax
import jax.numpy as jnp
from jax import lax
from jax.experimental import pallas as pl
from jax.experimental.pallas import tpu as pltpu
from jax.experimental.pallas import tpu_sc as plsc


def _atanh(z):
    return 0.5 * jnp.log((1.0 + z) / (1.0 - z))


C = 1.0
EPS = 1e-5
MIN_NORM = 1e-7
ALPHA = 0.1

N = 10000
E = 320000
D = 128
NP = 10240

NC = 2
NS = 16
DH = D // NC
EPT = E // NS
K = 80
NCH = EPT // K
RPT = N // NS


def _prep_body(x_ref, y0_ref, y1_ref, g_ref):
    x = x_ref[...]
    ss = jnp.sum(x * x, axis=-1, keepdims=True)
    g = 2.0 / jnp.clip(1.0 - C * ss, EPS, None)
    y = g * x
    y0_ref[...] = y[:, 0:DH]
    y1_ref[...] = y[:, DH:D]
    g_ref[...] = jnp.broadcast_to(g - 1.0, (x.shape[0], DH))


def _prep(x):
    BN = 1000
    return pl.pallas_call(
        _prep_body,
        grid=(N // BN,),
        in_specs=[pl.BlockSpec((BN, D), lambda i: (i, 0))],
        out_specs=[
            pl.BlockSpec((BN, DH), lambda i: (i, 0)),
            pl.BlockSpec((BN, DH), lambda i: (i, 0)),
            pl.BlockSpec((BN, DH), lambda i: (i, 0)),
        ],
        out_shape=[
            jax.ShapeDtypeStruct((N, DH), jnp.float32),
            jax.ShapeDtypeStruct((N, DH), jnp.float32),
            jax.ShapeDtypeStruct((N, DH), jnp.float32),
        ],
    )(x)


def _sc_body(y0_hbm, y1_hbm, gm1_hbm, src_hbm, dst_hbm, w_hbm, z2_hbm, z1_hbm,
             nom_hbm, den_hbm,
             src_v, dst_v, w_v, rows_v, gm1_v, den_acc_v, nom_sh, sem):
    cid = lax.axis_index("c")
    sid = lax.axis_index("s")

    pltpu.sync_copy(src_hbm.at[sid], src_v)
    pltpu.sync_copy(dst_hbm.at[sid], dst_v)
    pltpu.sync_copy(w_hbm.at[sid], w_v)

    pltpu.sync_copy(z2_hbm, nom_sh.at[pl.ds(sid * RPT, RPT)])

    @pl.when(cid == 0)
    def _():
        pltpu.sync_copy(gm1_hbm, gm1_v)
        pltpu.sync_copy(z1_hbm, den_acc_v)

    plsc.subcore_barrier()

    lanes = lax.iota(jnp.int32, 16)

    def chunk(c, carry):
        @pl.when(cid == 0)
        def _():
            pltpu.async_copy(y0_hbm.at[src_v.at[c]], rows_v, sem).wait()

        @pl.when(cid == 1)
        def _():
            pltpu.async_copy(y1_hbm.at[src_v.at[c]], rows_v, sem).wait()

        @pl.when(cid == 0)
        def _():
            for t in range(K // 16):
                sl = pl.ds(t * 16, 16)
                srcl = src_v[c, sl]
                dstl = dst_v[c, sl]
                wl = w_v[c, sl]
                val = wl * plsc.load_gather(gm1_v, [srcl])
                for i in range(16):
                    plsc.addupdate_scatter(den_acc_v, [dstl], val,
                                           mask=lanes == i)

        def edge(k, carry2):
            wk = plsc.load_gather(w_v, [jnp.full((16,), c, jnp.int32),
                                        jnp.full((16,), k, jnp.int32)])
            for j in range(DH // 16):
                sl = pl.ds(j * 16, 16)
                rows_v[k, sl] = rows_v[k, sl] * wk
            return carry2

        lax.fori_loop(0, K, edge, 0, unroll=4)

        pltpu.sync_copy(rows_v, nom_sh.at[dst_v.at[c]], add=True)
        return carry

    lax.fori_loop(0, NCH, chunk, 0)
    plsc.subcore_barrier()

    pltpu.sync_copy(nom_sh.at[pl.ds(sid * RPT, RPT)],
                    nom_hbm.at[cid, pl.ds(sid * RPT, RPT)])

    @pl.when(cid == 0)
    def _():
        pltpu.sync_copy(den_acc_v, den_hbm.at[sid])


def _sc_spmm(y0, y1, gm1, src, dst, w):
    mesh = plsc.VectorSubcoreMesh(core_axis_name="c", subcore_axis_name="s")
    f = pl.kernel(
        _sc_body,
        out_type=(
            jax.ShapeDtypeStruct((NC, N, DH), jnp.float32),
            jax.ShapeDtypeStruct((NS, NP), jnp.float32),
        ),
        mesh=mesh,
        scratch_types=[
            pltpu.VMEM((NCH, K), jnp.int32),
            pltpu.VMEM((NCH, K), jnp.int32),
            pltpu.VMEM((NCH, K), jnp.float32),
            pltpu.VMEM((K, DH), jnp.float32),
            pltpu.VMEM((N,), jnp.float32),
            pltpu.VMEM((NP,), jnp.float32),
            pltpu.VMEM_SHARED((N, DH), jnp.float32),
            pltpu.SemaphoreType.DMA,
        ],
        compiler_params=pltpu.CompilerParams(
            use_tc_tiling_on_sc=False, needs_layout_passes=False),
    )
    z2 = jnp.zeros((RPT, DH), jnp.float32)
    z1 = jnp.zeros((NP,), jnp.float32)
    return f(y0, y1, gm1, src, dst, w, z2, z1)


def _dred_body(d_ref, o_ref):
    o_ref[...] = jnp.sum(d_ref[...], axis=0, keepdims=True)


def _dred(dp):
    return pl.pallas_call(
        _dred_body,
        grid=(1,),
        in_specs=[pl.BlockSpec((NS, NP), lambda i: (0, 0))],
        out_specs=pl.BlockSpec((1, NP), lambda i: (0, 0)),
        out_shape=jax.ShapeDtypeStruct((1, NP), jnp.float32),
    )(dp)


def _project(x):
    sq = jnp.sum(x * x, axis=-1, keepdims=True)
    norm = jnp.maximum(jnp.sqrt(sq), MIN_NORM)
    maxnorm = (1.0 - 1e-5) / math.sqrt(C)
    return jnp.where(norm > maxnorm, x / norm * maxnorm, x)


def _mobius_half(x):
    sc = math.sqrt(C)
    sq = jnp.sum(x * x, axis=-1, keepdims=True)
    xn = jnp.maximum(jnp.sqrt(sq), MIN_NORM)
    z = jnp.clip(sc * xn, 0.0, 1.0 - 1e-5)
    t = jnp.tanh(0.5 * _atanh(z))
    return t * x / (sc * xn)


def _lam(x):
    return 2.0 / jnp.clip(1.0 - C * jnp.sum(x * x, axis=-1, keepdims=True), EPS, None)


def _fin_body(n0_ref, n1_ref, d_ref, h_ref, w_ref, b_ref, o_ref):
    nom = jnp.concatenate([n0_ref[...], n1_ref[...]], axis=1)
    den = d_ref[...]
    den = jnp.where(jnp.abs(den) < 1e-10, 1e-10, den)
    s = _project(_mobius_half(nom / den))

    h = h_ref[...]
    gs = _lam(s)
    gh = _lam(h)
    nom2 = (1.0 - ALPHA) * gs * s + ALPHA * gh * h
    den2 = (1.0 - ALPHA) * (gs - 1.0) + ALPHA * (gh - 1.0)
    den2 = jnp.where(jnp.abs(den2) < 1e-10, 1e-10, den2)
    s2 = _project(_mobius_half(nom2 / den2))

    sc = math.sqrt(C)
    sq = jnp.sum(s2 * s2, axis=-1, keepdims=True)
    xn = jnp.maximum(jnp.sqrt(sq), MIN_NORM)
    u = _atanh(jnp.clip(sc * xn, 0.0, 1.0 - 1e-5)) * s2 / (sc * xn)
    hl = lax.dot_general(u, w_ref[...], (((1,), (1,)), ((), ())),
                         preferred_element_type=jnp.float32) + b_ref[...]
    un = jnp.maximum(jnp.sqrt(jnp.sum(hl * hl, axis=-1, keepdims=True)), MIN_NORM)
    e = jnp.tanh(sc * un) * hl / (sc * un)
    o_ref[...] = _project(e)


def _finish(n0, n1, d, h_init, W, b2):
    BN = 1000
    return pl.pallas_call(
        _fin_body,
        grid=(N // BN,),
        in_specs=[
            pl.BlockSpec((BN, DH), lambda i: (i, 0)),
            pl.BlockSpec((BN, DH), lambda i: (i, 0)),
            pl.BlockSpec((BN, 1), lambda i: (i, 0)),
            pl.BlockSpec((BN, D), lambda i: (i, 0)),
            pl.BlockSpec((D, D), lambda i: (0, 0)),
            pl.BlockSpec((1, D), lambda i: (0, 0)),
        ],
        out_specs=pl.BlockSpec((BN, D), lambda i: (i, 0)),
        out_shape=jax.ShapeDtypeStruct((N, D), jnp.float32),
    )(n0, n1, d, h_init, W, b2)


@jax.jit
def kernel(input, edge_index, edge_weight, h_init, W, b):
    y0, y1, g2d = _prep(input)
    gm1 = g2d[:, 0]
    src = edge_index[0].reshape(NS, NCH, K)
    dst = edge_index[1].reshape(NS, NCH, K)
    w = edge_weight.reshape(NS, NCH, K)
    nom, denp = _sc_spmm(y0, y1, gm1, src, dst, w)
    den = _dred(denp)[0, :N].reshape(N, 1)
    return _finish(nom[0], nom[1], den, h_init, W, b.reshape(1, D))


# speedup vs baseline: 10.6317x; 1.5607x over previous
import math

import jax
import jax.numpy as jnp
from jax import lax
from jax.experimental import pallas as pl
from jax.experimental.pallas import tpu as pltpu
from jax.experimental.pallas import tpu_sc as plsc


def _atanh(z):
    return 0.5 * jnp.log((1.0 + z) / (1.0 - z))


C = 1.0
EPS = 1e-5
MIN_NORM = 1e-7
ALPHA = 0.1

N = 10000
E = 320000
D = 128
NP = 10240

NC = 2
NS = 16
DH = D // NC
EPT = E // NS
K = 80
NCH = EPT // K
PAIRS = NCH // 2
RPT = N // NS


def _prep_body(x_ref, y0_ref, y1_ref, g_ref):
    x = x_ref[...]
    ss = jnp.sum(x * x, axis=-1, keepdims=True)
    g = 2.0 / jnp.clip(1.0 - C * ss, EPS, None)
    y = g * x
    y0_ref[...] = y[:, 0:DH]
    y1_ref[...] = y[:, DH:D]
    g_ref[...] = jnp.broadcast_to(g - 1.0, (x.shape[0], DH))


def _prep(x):
    BN = 1000
    return pl.pallas_call(
        _prep_body,
        grid=(N // BN,),
        in_specs=[pl.BlockSpec((BN, D), lambda i: (i, 0))],
        out_specs=[
            pl.BlockSpec((BN, DH), lambda i: (i, 0)),
            pl.BlockSpec((BN, DH), lambda i: (i, 0)),
            pl.BlockSpec((BN, DH), lambda i: (i, 0)),
        ],
        out_shape=[
            jax.ShapeDtypeStruct((N, DH), jnp.float32),
            jax.ShapeDtypeStruct((N, DH), jnp.float32),
            jax.ShapeDtypeStruct((N, DH), jnp.float32),
        ],
    )(x)


def _sc_body(y0_hbm, y1_hbm, gm1_hbm, src_hbm, dst_hbm, w_hbm, z2_hbm, z1_hbm,
             nom_hbm, den_hbm,
             src_v, dst_v, w_v, rows_a, rows_b, gm1_v, den_acc_v, nom_sh,
             gsa, gsb, ssa, ssb):
    cid = lax.axis_index("c")
    sid = lax.axis_index("s")

    pltpu.sync_copy(src_hbm.at[sid], src_v)
    pltpu.sync_copy(dst_hbm.at[sid], dst_v)
    pltpu.sync_copy(w_hbm.at[sid], w_v)

    pltpu.sync_copy(z2_hbm, nom_sh.at[pl.ds(sid * RPT, RPT)])

    @pl.when(cid == 0)
    def _():
        pltpu.sync_copy(gm1_hbm, gm1_v)
        pltpu.sync_copy(z1_hbm, den_acc_v)

    plsc.subcore_barrier()

    lanes = lax.iota(jnp.int32, 16)

    def gather_start(c, buf, sem):
        @pl.when(cid == 0)
        def _():
            pltpu.async_copy(y0_hbm.at[src_v.at[c]], buf, sem)

        @pl.when(cid == 1)
        def _():
            pltpu.async_copy(y1_hbm.at[src_v.at[c]], buf, sem)

    def gather_wait(buf, sem):
        pltpu.make_async_copy(y0_hbm.at[src_v.at[0]], buf, sem).wait()

    def scatter_start(c, buf, sem):
        pltpu.async_copy(buf, nom_sh.at[dst_v.at[c]], sem, add=True)

    def scatter_wait(buf, sem):
        pltpu.make_async_copy(buf, nom_sh.at[dst_v.at[0]], sem).wait()

    def proc(c, buf):
        @pl.when(cid == 0)
        def _():
            for t in range(K // 16):
                sl = pl.ds(t * 16, 16)
                srcl = src_v[c, sl]
                dstl = dst_v[c, sl]
                wl = w_v[c, sl]
                val = wl * plsc.load_gather(gm1_v, [srcl])
                for i in range(16):
                    plsc.addupdate_scatter(den_acc_v, [dstl], val,
                                           mask=lanes == i)

        def edge(k, carry2):
            wk = plsc.load_gather(w_v, [jnp.full((16,), c, jnp.int32),
                                        jnp.full((16,), k, jnp.int32)])
            for j in range(DH // 16):
                sl = pl.ds(j * 16, 16)
                buf[k, sl] = buf[k, sl] * wk
            return carry2

        lax.fori_loop(0, K, edge, 0, unroll=4)

    gather_start(0, rows_a, gsa)

    def pair(i, carry):
        c0 = 2 * i
        c1 = 2 * i + 1
        gather_wait(rows_a, gsa)

        @pl.when(i > 0)
        def _():
            scatter_wait(rows_b, ssb)

        gather_start(c1, rows_b, gsb)
        proc(c0, rows_a)
        scatter_start(c0, rows_a, ssa)

        gather_wait(rows_b, gsb)
        scatter_wait(rows_a, ssa)

        @pl.when(i < PAIRS - 1)
        def _():
            gather_start(c0 + 2, rows_a, gsa)

        proc(c1, rows_b)
        scatter_start(c1, rows_b, ssb)
        return carry

    lax.fori_loop(0, PAIRS, pair, 0)
    scatter_wait(rows_b, ssb)
    plsc.subcore_barrier()

    pltpu.sync_copy(nom_sh.at[pl.ds(sid * RPT, RPT)],
                    nom_hbm.at[cid, pl.ds(sid * RPT, RPT)])

    @pl.when(cid == 0)
    def _():
        pltpu.sync_copy(den_acc_v, den_hbm.at[sid])


def _sc_spmm(y0, y1, gm1, src, dst, w):
    mesh = plsc.VectorSubcoreMesh(core_axis_name="c", subcore_axis_name="s")
    f = pl.kernel(
        _sc_body,
        out_type=(
            jax.ShapeDtypeStruct((NC, N, DH), jnp.float32),
            jax.ShapeDtypeStruct((NS, NP), jnp.float32),
        ),
        mesh=mesh,
        scratch_types=[
            pltpu.VMEM((NCH, K), jnp.int32),
            pltpu.VMEM((NCH, K), jnp.int32),
            pltpu.VMEM((NCH, K), jnp.float32),
            pltpu.VMEM((K, DH), jnp.float32),
            pltpu.VMEM((K, DH), jnp.float32),
            pltpu.VMEM((N,), jnp.float32),
            pltpu.VMEM((NP,), jnp.float32),
            pltpu.VMEM_SHARED((N, DH), jnp.float32),
            pltpu.SemaphoreType.DMA,
            pltpu.SemaphoreType.DMA,
            pltpu.SemaphoreType.DMA,
            pltpu.SemaphoreType.DMA,
        ],
        compiler_params=pltpu.CompilerParams(
            use_tc_tiling_on_sc=False, needs_layout_passes=False),
    )
    z2 = jnp.zeros((RPT, DH), jnp.float32)
    z1 = jnp.zeros((NP,), jnp.float32)
    return f(y0, y1, gm1, src, dst, w, z2, z1)


def _dred_body(d_ref, o_ref):
    o_ref[...] = jnp.sum(d_ref[...], axis=0, keepdims=True)


def _dred(dp):
    return pl.pallas_call(
        _dred_body,
        grid=(1,),
        in_specs=[pl.BlockSpec((NS, NP), lambda i: (0, 0))],
        out_specs=pl.BlockSpec((1, NP), lambda i: (0, 0)),
        out_shape=jax.ShapeDtypeStruct((1, NP), jnp.float32),
    )(dp)


def _project(x):
    sq = jnp.sum(x * x, axis=-1, keepdims=True)
    norm = jnp.maximum(jnp.sqrt(sq), MIN_NORM)
    maxnorm = (1.0 - 1e-5) / math.sqrt(C)
    return jnp.where(norm > maxnorm, x / norm * maxnorm, x)


def _mobius_half(x):
    sc = math.sqrt(C)
    sq = jnp.sum(x * x, axis=-1, keepdims=True)
    xn = jnp.maximum(jnp.sqrt(sq), MIN_NORM)
    z = jnp.clip(sc * xn, 0.0, 1.0 - 1e-5)
    t = jnp.tanh(0.5 * _atanh(z))
    return t * x / (sc * xn)


def _lam(x):
    return 2.0 / jnp.clip(1.0 - C * jnp.sum(x * x, axis=-1, keepdims=True), EPS, None)


def _fin_body(n0_ref, n1_ref, d_ref, h_ref, w_ref, b_ref, o_ref):
    nom = jnp.concatenate([n0_ref[...], n1_ref[...]], axis=1)
    den = d_ref[...]
    den = jnp.where(jnp.abs(den) < 1e-10, 1e-10, den)
    s = _project(_mobius_half(nom / den))

    h = h_ref[...]
    gs = _lam(s)
    gh = _lam(h)
    nom2 = (1.0 - ALPHA) * gs * s + ALPHA * gh * h
    den2 = (1.0 - ALPHA) * (gs - 1.0) + ALPHA * (gh - 1.0)
    den2 = jnp.where(jnp.abs(den2) < 1e-10, 1e-10, den2)
    s2 = _project(_mobius_half(nom2 / den2))

    sc = math.sqrt(C)
    sq = jnp.sum(s2 * s2, axis=-1, keepdims=True)
    xn = jnp.maximum(jnp.sqrt(sq), MIN_NORM)
    u = _atanh(jnp.clip(sc * xn, 0.0, 1.0 - 1e-5)) * s2 / (sc * xn)
    hl = lax.dot_general(u, w_ref[...], (((1,), (1,)), ((), ())),
                         preferred_element_type=jnp.float32) + b_ref[...]
    un = jnp.maximum(jnp.sqrt(jnp.sum(hl * hl, axis=-1, keepdims=True)), MIN_NORM)
    e = jnp.tanh(sc * un) * hl / (sc * un)
    o_ref[...] = _project(e)


def _finish(n0, n1, d, h_init, W, b2):
    BN = 1000
    return pl.pallas_call(
        _fin_body,
        grid=(N // BN,),
        in_specs=[
            pl.BlockSpec((BN, DH), lambda i: (i, 0)),
            pl.BlockSpec((BN, DH), lambda i: (i, 0)),
            pl.BlockSpec((BN, 1), lambda i: (i, 0)),
            pl.BlockSpec((BN, D), lambda i: (i, 0)),
            pl.BlockSpec((D, D), lambda i: (0, 0)),
            pl.BlockSpec((1, D), lambda i: (0, 0)),
        ],
        out_specs=pl.BlockSpec((BN, D), lambda i: (i, 0)),
        out_shape=jax.ShapeDtypeStruct((N, D), jnp.float32),
    )(n0, n1, d, h_init, W, b2)


@jax.jit
def kernel(input, edge_index, edge_weight, h_init, W, b):
    y0, y1, g2d = _prep(input)
    gm1 = g2d[:, 0]
    src = edge_index[0].reshape(NS, NCH, K)
    dst = edge_index[1].reshape(NS, NCH, K)
    w = edge_weight.reshape(NS, NCH, K)
    nom, denp = _sc_spmm(y0, y1, gm1, src, dst, w)
    den = _dred(denp)[0, :N].reshape(N, 1)
    return _finish(nom[0], nom[1], den, h_init, W, b.reshape(1, D))
